# Initial kernel scaffold; baseline (speedup 1.0000x reference)
import jax
import jax.numpy as jnp
from jax.experimental import pallas as pl


def kernel(x, W_ih_enc, W_hh_enc, b_ih_enc, b_hh_enc, W_ih_dec, W_hh_dec, b_ih_dec, b_hh_dec, W_q, b_q, end_node_embed, start_token):
    raise NotImplementedError("write your pallas kernel here")


# speedup vs baseline: 8.4382x; 8.4382x over previous
import functools

import jax
import jax.numpy as jnp
from jax.experimental import pallas as pl
from jax.experimental.pallas import tpu as pltpu

HID = 128
INP = 3
MAX_STEPS = 10
B, N = 16, 4096
N_EXT = N + 1
N_PAD = 4224
CHUNK = 128
N_CHUNKS_ENC = N // CHUNK
N_CHUNKS_DEC = N_PAD // CHUNK
NEG = -3.0e38

def _sig(x):
    return jax.nn.sigmoid(x)


def _fused_kernel(x_hbm, wih_enc_t, whh_enc_t, b_enc, wih_dec_t, whh_dec_t,
                  b_dec, wq_t, b_q, end_node, start_tok,
                  logits_out, idx_out,
                  enc_scr, xbuf, gxbuf, logits_scr, dma_sem):
    f32 = jnp.float32

    def enc_chunk(cidx, carry):
        h, c = carry
        cp = pltpu.make_async_copy(
            x_hbm.at[pl.ds(cidx * CHUNK, CHUNK)], xbuf, dma_sem)
        cp.start()
        cp.wait()
        gx = b_enc[0][None, None, :]
        for j in range(INP):
            xv = xbuf[:, :, j].astype(jnp.bfloat16).astype(f32)
            wv = wih_enc_t[j].astype(f32)
            gx = gx + xv[:, :, None] * wv[None, None, :]
        gxbuf[...] = gx

        def step(t, hc):
            h, c = hc
            gates = gxbuf[t] + jax.lax.dot_general(
                h.astype(jnp.bfloat16), whh_enc_t[...],
                (((1,), (0,)), ((), ())), preferred_element_type=f32)
            i = gates[:, 0:HID]
            f = gates[:, HID:2 * HID]
            g = gates[:, 2 * HID:3 * HID]
            o = gates[:, 3 * HID:4 * HID]
            c2 = _sig(f) * c + _sig(i) * jnp.tanh(g)
            h2 = _sig(o) * jnp.tanh(c2)
            enc_scr[cidx * CHUNK + t] = h2.astype(jnp.bfloat16)
            return (h2, c2)

        return jax.lax.fori_loop(0, CHUNK, step, (h, c))

    h0 = jnp.zeros((B, HID), f32)
    c0 = jnp.zeros((B, HID), f32)
    h_fin, c_fin = jax.lax.fori_loop(0, N_CHUNKS_ENC, enc_chunk, (h0, c0))

    enc_scr[N] = jnp.broadcast_to(end_node[0], (B, HID)).astype(jnp.bfloat16)
    enc_scr[pl.ds(N_EXT, N_PAD - N_EXT)] = jnp.zeros(
        (N_PAD - N_EXT, B, HID), jnp.bfloat16)

    lane2 = jax.lax.broadcasted_iota(jnp.int32, (B, N_PAD), 1)
    lane128 = jax.lax.broadcasted_iota(jnp.int32, (B, HID), 1)

    hd, cd = h_fin, c_fin
    gates_x = (
        jax.lax.dot_general(
            jnp.broadcast_to(start_tok[0], (B, 3 * HID)).astype(jnp.bfloat16),
            wih_dec_t[...],
            (((1,), (0,)), ((), ())), preferred_element_type=f32)
        + b_dec[0][None, :])

    for t in range(MAX_STEPS):
        gates = gates_x + jax.lax.dot_general(
            hd.astype(jnp.bfloat16), whh_dec_t[...],
            (((1,), (0,)), ((), ())), preferred_element_type=f32)
        i = gates[:, 0:HID]
        f = gates[:, HID:2 * HID]
        g = gates[:, 2 * HID:3 * HID]
        o = gates[:, 3 * HID:4 * HID]
        cd = _sig(f) * cd + _sig(i) * jnp.tanh(g)
        hd = _sig(o) * jnp.tanh(cd)
        q = jax.lax.dot_general(
            hd.astype(jnp.bfloat16), wq_t[...], (((1,), (0,)), ((), ())),
            preferred_element_type=f32) + b_q[0][None, :]
        qb = q.astype(jnp.bfloat16).astype(f32)

        def logit_chunk(cidx, _):
            chunk = enc_scr[pl.ds(cidx * CHUNK, CHUNK)].astype(f32)
            p = jnp.sum(chunk * qb[None, :, :], axis=2)
            logits_scr[:, pl.ds(cidx * CHUNK, CHUNK)] = p.T
            return 0

        jax.lax.fori_loop(0, N_CHUNKS_DEC, logit_chunk, 0)

        logits = logits_scr[...]
        logits_out[t] = logits

        work = jnp.where(lane2 < N_EXT, logits, NEG)
        idxs = []
        for _k in range(3):
            m = jnp.max(work, axis=1, keepdims=True)
            cand = jnp.where(work == m, lane2, jnp.int32(N_PAD))
            ik = jnp.min(cand, axis=1, keepdims=True)
            idxs.append(ik)
            work = jnp.where(lane2 == ik, NEG, work)

        idx_comb = (
            jnp.where(lane128 == 0, idxs[0], 0)
            + jnp.where(lane128 == 1, idxs[1], 0)
            + jnp.where(lane128 == 2, idxs[2], 0))
        idx_out[t] = idx_comb

        ik_rows = [ik.T for ik in idxs]

        def gather_chunk(cidx, accs):
            chunk = enc_scr[pl.ds(cidx * CHUNK, CHUNK)].astype(f32)
            niota = (jax.lax.broadcasted_iota(jnp.int32, (CHUNK, B), 0)
                     + cidx * CHUNK)
            new = []
            for k in range(3):
                msk = (niota == ik_rows[k]).astype(f32)
                new.append(accs[k]
                           + jnp.sum(chunk * msk[:, :, None], axis=0))
            return tuple(new)

        z = jnp.zeros((B, HID), f32)
        v0, v1, v2 = jax.lax.fori_loop(
            0, N_CHUNKS_DEC, gather_chunk, (z, z, z))

        gates_x = (
            jax.lax.dot_general(
                v0.astype(jnp.bfloat16), wih_dec_t[0:HID],
                (((1,), (0,)), ((), ())), preferred_element_type=f32)
            + jax.lax.dot_general(
                v1.astype(jnp.bfloat16), wih_dec_t[HID:2 * HID],
                (((1,), (0,)), ((), ())), preferred_element_type=f32)
            + jax.lax.dot_general(
                v2.astype(jnp.bfloat16), wih_dec_t[2 * HID:3 * HID],
                (((1,), (0,)), ((), ())), preferred_element_type=f32)
            + b_dec[0][None, :])


@functools.partial(jax.jit, static_argnames=("interpret",))
def kernel(x, W_ih_enc, W_hh_enc, b_ih_enc, b_hh_enc,
           W_ih_dec, W_hh_dec, b_ih_dec, b_hh_dec,
           W_q, b_q, end_node_embed, start_token, interpret=False):
    f32 = jnp.float32
    x_tm = jnp.swapaxes(x, 0, 1)

    logits_pad, idx_pad = pl.pallas_call(
        _fused_kernel,
        out_shape=[
            jax.ShapeDtypeStruct((MAX_STEPS, B, N_PAD), f32),
            jax.ShapeDtypeStruct((MAX_STEPS, B, HID), jnp.int32),
        ],
        in_specs=[pl.BlockSpec(memory_space=pl.ANY)]
        + [pl.BlockSpec()] * 10,
        out_specs=[pl.BlockSpec(), pl.BlockSpec()],
        scratch_shapes=[
            pltpu.VMEM((N_PAD, B, HID), jnp.bfloat16),
            pltpu.VMEM((CHUNK, B, INP), f32),
            pltpu.VMEM((CHUNK, B, 4 * HID), f32),
            pltpu.VMEM((B, N_PAD), f32),
            pltpu.SemaphoreType.DMA,
        ],
        interpret=interpret,
    )(
        x_tm,
        W_ih_enc.T.astype(jnp.bfloat16), W_hh_enc.T.astype(jnp.bfloat16),
        (b_ih_enc + b_hh_enc)[None, :],
        W_ih_dec.T.astype(jnp.bfloat16), W_hh_dec.T.astype(jnp.bfloat16),
        (b_ih_dec + b_hh_dec)[None, :],
        W_q.T.astype(jnp.bfloat16), b_q[None, :], end_node_embed, start_token,
    )

    return logits_pad[:, :, :N_EXT], idx_pad[:, :, :3]
